# Initial kernel scaffold; baseline (speedup 1.0000x reference)
import jax
import jax.numpy as jnp
from jax.experimental import pallas as pl


def kernel(ui_indices, ui_values, user_id_emb, item_id_emb, image_feats, text_feats, W_img, b_img, W_txt, b_txt):
    raise NotImplementedError("write your pallas kernel here")


# speedup vs baseline: 1.9829x; 1.9829x over previous
import functools

import jax
import jax.numpy as jnp
from jax import lax
from jax.experimental import pallas as pl
from jax.experimental.pallas import tpu as pltpu
from jax.experimental.pallas import tpu_sc as plsc

_N = 50000
_D = 64
_E = 800000
_W = 32
_CH = 80
_NTILE = 16
_EPT = _E // _NTILE
_NCH = _EPT // _CH
_DR = 400
_NDCH = _N // _DR
_DPT = (_NDCH + _NTILE - 1) // _NTILE


def _make_spmm(S):
    mesh = plsc.VectorSubcoreMesh(core_axis_name="c", subcore_axis_name="s")

    @functools.partial(
        pl.kernel,
        mesh=mesh,
        compiler_params=pltpu.CompilerParams(use_tc_tiling_on_sc=False),
        out_type=jax.ShapeDtypeStruct((S * _N, _W), jnp.float32),
        scratch_types=[
            pltpu.VMEM((_CH,), jnp.int32),
            pltpu.VMEM((_CH,), jnp.int32),
            pltpu.VMEM((_CH,), jnp.int32),
            pltpu.VMEM((_CH,), jnp.float32),
            pltpu.VMEM((_CH, _W), jnp.float32),
            pltpu.VMEM((_DR, _W), jnp.float32),
            pltpu.VMEM((_DR, _W), jnp.float32),
            pltpu.VMEM_SHARED((_N, _W), jnp.float32),
            pltpu.SemaphoreType.DMA,
        ],
    )
    def spmm(tab_hbm, gidx_hbm, sidx_hbm, val_hbm, out_hbm,
             gi_v, ga_v, si_v, va_v, rows_v, zero_v, ostg_v, acc_sh, sem):
        c = lax.axis_index("c")
        t = lax.axis_index("s")

        def zbody(i, carry):
            zero_v[i, pl.ds(0, 16)] = jnp.zeros((16,), jnp.float32)
            zero_v[i, pl.ds(16, 16)] = jnp.zeros((16,), jnp.float32)
            return carry

        lax.fori_loop(0, _DR, zbody, 0)

        for sl in range(S // 2):
            slice_id = c + 2 * sl
            soff = slice_id * _N
            for j in range(_DPT):
                ci = t + _NTILE * j

                @pl.when(ci < _NDCH)
                def _(ci=ci):
                    pltpu.sync_copy(zero_v, acc_sh.at[pl.ds(ci * _DR, _DR)])

            plsc.subcore_barrier()

            ebase = t * _EPT

            def chunk_body(k, carry):
                off = ebase + k * _CH
                pltpu.sync_copy(gidx_hbm.at[pl.ds(off, _CH)], gi_v)
                pltpu.sync_copy(sidx_hbm.at[pl.ds(off, _CH)], si_v)
                pltpu.sync_copy(val_hbm.at[pl.ds(off, _CH)], va_v)
                for j in range(_CH // 16):
                    ga_v[pl.ds(j * 16, 16)] = gi_v[pl.ds(j * 16, 16)] + soff
                pltpu.async_copy(tab_hbm.at[ga_v], rows_v, sem).wait()
                for g in range(_CH // 16):
                    vvec = va_v[pl.ds(g * 16, 16)]
                    for j in range(16):
                        e = g * 16 + j
                        v = vvec[j]
                        rows_v[e, pl.ds(0, 16)] = rows_v[e, pl.ds(0, 16)] * v
                        rows_v[e, pl.ds(16, 16)] = rows_v[e, pl.ds(16, 16)] * v
                pltpu.sync_copy(rows_v, acc_sh.at[si_v], add=True)
                return carry

            lax.fori_loop(0, _NCH, chunk_body, 0)
            plsc.subcore_barrier()
            for j in range(_DPT):
                ci = t + _NTILE * j

                @pl.when(ci < _NDCH)
                def _(ci=ci):
                    r0 = ci * _DR
                    pltpu.sync_copy(acc_sh.at[pl.ds(r0, _DR)], ostg_v)
                    pltpu.sync_copy(ostg_v, out_hbm.at[pl.ds(soff + r0, _DR)])

    return spmm


_spmm6 = _make_spmm(6)
_spmm2 = _make_spmm(2)


def _to_slices(x, s):
    return x.reshape(_N, s, _W).transpose(1, 0, 2).reshape(s * _N, _W)


def _from_slices(y, s):
    return y.reshape(s, _N, _W).transpose(1, 0, 2).reshape(_N, s * _W)


_BR = 2000


def _proj_body(x_ref, w_ref, b_ref, o_ref):
    o_ref[...] = (
        jnp.dot(x_ref[...], w_ref[...], preferred_element_type=jnp.float32)
        + b_ref[...]
    )


def _proj(x, w, b):
    n, k = x.shape
    dout = w.shape[0]
    return pl.pallas_call(
        _proj_body,
        grid=(n // _BR,),
        in_specs=[
            pl.BlockSpec((_BR, k), lambda i: (i, 0)),
            pl.BlockSpec((k, dout), lambda i: (0, 0)),
            pl.BlockSpec((1, dout), lambda i: (0, 0)),
        ],
        out_specs=pl.BlockSpec((_BR, dout), lambda i: (i, 0)),
        out_shape=jax.ShapeDtypeStruct((n, dout), jnp.float32),
    )(x, w.T, b.reshape(1, dout))


def _softmax_body(x_ref, o_ref):
    x = x_ref[...]
    m = jnp.max(x, axis=1, keepdims=True)
    e = jnp.exp(x - m)
    o_ref[...] = e / jnp.sum(e, axis=1, keepdims=True)


def _softmax(x):
    return pl.pallas_call(
        _softmax_body,
        grid=(_N // _BR,),
        in_specs=[pl.BlockSpec((_BR, _D), lambda i: (i, 0))],
        out_specs=pl.BlockSpec((_BR, _D), lambda i: (i, 0)),
        out_shape=jax.ShapeDtypeStruct((_N, _D), jnp.float32),
    )(x)


def _l2n(x):
    n = jnp.sqrt(jnp.sum(x * x, axis=1, keepdims=True))
    return x / jnp.clip(n, 1e-12, None)


def _combine_user_body(e0_ref, e1_ref, e2_ref, m1_ref, m2_ref, o_ref):
    mean = (e0_ref[...] + e1_ref[...] + e2_ref[...]) * (1.0 / 3.0)
    o_ref[...] = mean + 0.55 * _l2n(m1_ref[...]) + 0.55 * _l2n(m2_ref[...])


def _combine_item_body(e0_ref, e1_ref, pre2_ref, m1_ref, m2_ref, o_ref):
    x = pre2_ref[...]
    m = jnp.max(x, axis=1, keepdims=True)
    e = jnp.exp(x - m)
    e2 = e / jnp.sum(e, axis=1, keepdims=True)
    mean = (e0_ref[...] + e1_ref[...] + e2) * (1.0 / 3.0)
    o_ref[...] = mean + 0.55 * _l2n(m1_ref[...]) + 0.55 * _l2n(m2_ref[...])


def _combine(body, e0, e1, e2, m1, m2):
    spec = pl.BlockSpec((_BR, _D), lambda i: (i, 0))
    return pl.pallas_call(
        body,
        grid=(_N // _BR,),
        in_specs=[spec] * 5,
        out_specs=spec,
        out_shape=jax.ShapeDtypeStruct((_N, _D), jnp.float32),
    )(e0, e1, e2, m1, m2)


def kernel(ui_indices, ui_values, user_id_emb, item_id_emb, image_feats,
           text_feats, W_img, b_img, W_txt, b_txt):
    rows = ui_indices[0].astype(jnp.int32)
    cols = ui_indices[1].astype(jnp.int32)
    vals = ui_values.astype(jnp.float32)

    image_feat = _proj(image_feats, W_img, b_img)
    text_feat = _proj(text_feats, W_txt, b_txt)

    item_in = _to_slices(
        jnp.concatenate([image_feat, text_feat, item_id_emb], axis=1), 6)
    user_out = _spmm6(item_in, cols, rows, vals)
    item_out = _spmm6(user_out, rows, cols, vals)

    user_mat = _from_slices(user_out, 6)
    item_mat = _from_slices(item_out, 6)
    image_user_feats = user_mat[:, 0:64]
    text_user_feats = user_mat[:, 64:128]
    image_item_feats = item_mat[:, 0:64]
    text_item_feats = item_mat[:, 64:128]
    u_g1 = user_mat[:, 128:192]
    i_g1 = item_mat[:, 128:192]

    i_g1_slices = item_out[4 * _N:]
    pre_u2 = _from_slices(_spmm2(i_g1_slices, cols, rows, vals), 2)
    u_g2 = _softmax(pre_u2)
    pre_i2 = _from_slices(_spmm2(_to_slices(u_g2, 2), rows, cols, vals), 2)

    u_final = _combine(_combine_user_body, user_id_emb, u_g1, u_g2,
                       image_user_feats, text_user_feats)
    i_final = _combine(_combine_item_body, item_id_emb, i_g1, pre_i2,
                       image_item_feats, text_item_feats)

    prompt_user = jnp.zeros((_N, _D), jnp.float32)
    prompt_item = jnp.zeros((_N, _D), jnp.float32)
    return (u_final, i_final, image_item_feats, text_item_feats,
            image_user_feats, text_user_feats, u_final, i_final,
            prompt_user, prompt_item, 0.0)


# speedup vs baseline: 3.0440x; 1.5351x over previous
import functools

import jax
import jax.numpy as jnp
from jax import lax
from jax.experimental import pallas as pl
from jax.experimental.pallas import tpu as pltpu
from jax.experimental.pallas import tpu_sc as plsc

_N = 50000
_D = 64
_E = 800000
_W = 32
_AW = 16
_CH = 128
_SUB = 4
_NTILE = 16
_SCE = _SUB * _CH
_NSC = (_E // _NTILE + _SCE - 1) // _SCE
_EPT = _NSC * _SCE
_EP = _EPT * _NTILE
_ROWS_PT = _EPT // _CH
_DR = 400
_NDCH = _N // _DR
_DPT = (_NDCH + _NTILE - 1) // _NTILE


def _make_spmm(S):
    mesh = plsc.VectorSubcoreMesh(core_axis_name="c", subcore_axis_name="s")

    @functools.partial(
        pl.kernel,
        mesh=mesh,
        compiler_params=pltpu.CompilerParams(use_tc_tiling_on_sc=False),
        out_type=jax.ShapeDtypeStruct((S * _N, _W), jnp.float32),
        scratch_types=[
            pltpu.VMEM((2, _SUB, _CH), jnp.int32),
            pltpu.VMEM((2, _SUB, _CH), jnp.int32),
            pltpu.VMEM((2, _SUB, _CH), jnp.int32),
            pltpu.VMEM((2, _SUB, _CH), jnp.float32),
            pltpu.VMEM((2, _CH, _W), jnp.float32),
            pltpu.VMEM((_CH, _AW), jnp.float32),
            pltpu.VMEM((_DR, _AW), jnp.float32),
            pltpu.VMEM((_DR, _AW), jnp.float32),
            pltpu.VMEM_SHARED((_N, _AW), jnp.float32),
            pltpu.SemaphoreType.DMA,
            pltpu.SemaphoreType.DMA,
            pltpu.SemaphoreType.DMA,
            pltpu.SemaphoreType.DMA,
        ],
    )
    def spmm(tab_hbm, gidx_hbm, sidx_hbm, val_hbm, out_hbm,
             gi_v, ga_v, si_v, va_v, rows_v, sbuf_v, zero_v, ostg_v, acc_sh,
             sem_i0, sem_i1, sem_g0, sem_g1):
        c = lax.axis_index("c")
        t = lax.axis_index("s")
        hoff = c * _AW
        sem_i = (sem_i0, sem_i1)
        sem_g = (sem_g0, sem_g1)

        def zbody(i, carry):
            zero_v[i, pl.ds(0, 16)] = jnp.zeros((16,), jnp.float32)
            return carry

        lax.fori_loop(0, _DR, zbody, 0)

        rbase = t * _ROWS_PT

        def fire_idx(s, p):
            r = rbase + s * _SUB
            pltpu.async_copy(gidx_hbm.at[pl.ds(r, _SUB)], gi_v.at[p], sem_i[p])
            pltpu.async_copy(sidx_hbm.at[pl.ds(r, _SUB)], si_v.at[p], sem_i[p])
            pltpu.async_copy(val_hbm.at[pl.ds(r, _SUB)], va_v.at[p], sem_i[p])

        def wait_idx(s, p):
            r = rbase + s * _SUB
            pltpu.make_async_copy(gidx_hbm.at[pl.ds(r, _SUB)], gi_v.at[p],
                                  sem_i[p]).wait()
            pltpu.make_async_copy(sidx_hbm.at[pl.ds(r, _SUB)], si_v.at[p],
                                  sem_i[p]).wait()
            pltpu.make_async_copy(val_hbm.at[pl.ds(r, _SUB)], va_v.at[p],
                                  sem_i[p]).wait()

        def process(s, p, soff):
            wait_idx(s, p)
            for j in range(_SUB):
                for g in range(_CH // 16):
                    ga_v[p, j, pl.ds(g * 16, 16)] = (
                        gi_v[p, j, pl.ds(g * 16, 16)] + soff)
            pltpu.async_copy(tab_hbm.at[ga_v.at[p, 0]], rows_v.at[0],
                             sem_g[0])
            for j in range(_SUB):
                rp = j % 2
                if j + 1 < _SUB:
                    pltpu.async_copy(tab_hbm.at[ga_v.at[p, j + 1]],
                                     rows_v.at[1 - rp], sem_g[1 - rp])
                pltpu.make_async_copy(tab_hbm.at[ga_v.at[p, j]],
                                      rows_v.at[rp], sem_g[rp]).wait()

                def scale_body(g, carry, j=j, rp=rp):
                    b = g * 16
                    vvec = va_v[p, j, pl.ds(b, 16)]
                    for i in range(16):
                        v = vvec[i]
                        sbuf_v[b + i, pl.ds(0, 16)] = (
                            rows_v[rp, b + i, pl.ds(hoff, 16)] * v)
                    return carry

                lax.fori_loop(0, _CH // 16, scale_body, 0)
                pltpu.sync_copy(sbuf_v, acc_sh.at[si_v.at[p, j]],
                                add=True)

        def slice_body(sl, carry):
            soff = sl * _N
            for j in range(_DPT):
                ci = t + _NTILE * j

                @pl.when(ci < _NDCH)
                def _(ci=ci):
                    pltpu.sync_copy(zero_v, acc_sh.at[pl.ds(ci * _DR, _DR)])

            plsc.subcore_barrier()

            fire_idx(0, 0)

            def sc_pair(k, carry2):
                s0 = 2 * k
                fire_idx(s0 + 1, 1)
                process(s0, 0, soff)

                @pl.when(s0 + 2 < _NSC)
                def _():
                    fire_idx(s0 + 2, 0)

                process(s0 + 1, 1, soff)
                return carry2

            lax.fori_loop(0, _NSC // 2, sc_pair, 0)
            plsc.subcore_barrier()
            for j in range(_DPT):
                ci = t + _NTILE * j

                @pl.when(ci < _NDCH)
                def _(ci=ci):
                    r0 = ci * _DR
                    pltpu.sync_copy(acc_sh.at[pl.ds(r0, _DR)], ostg_v)
                    pltpu.sync_copy(
                        ostg_v,
                        out_hbm.at[pl.ds(soff + r0, _DR), pl.ds(hoff, _AW)])

            return carry

        lax.fori_loop(0, S, slice_body, 0)

    return spmm


_spmm6 = _make_spmm(6)
_spmm2 = _make_spmm(2)


_BR = 2000
_NB = _N // _BR


def _proj_body(x_ref, w_ref, b_ref, o_ref):
    o_ref[...] = (
        jax.lax.dot_general(x_ref[...], w_ref[...],
                            (((1,), (1,)), ((), ())),
                            preferred_element_type=jnp.float32)
        + b_ref[...]
    )


def _proj(x, w, b):
    n, k = x.shape
    return pl.pallas_call(
        _proj_body,
        grid=(_NB,),
        in_specs=[
            pl.BlockSpec((_BR, k), lambda i: (i, 0)),
            pl.BlockSpec((_D, k), lambda i: (0, 0)),
            pl.BlockSpec((1, _D), lambda i: (0, 0)),
        ],
        out_specs=pl.BlockSpec((_BR, _D), lambda i: (i, 0)),
        out_shape=jax.ShapeDtypeStruct((_N, _D), jnp.float32),
    )(x, w, b.reshape(1, _D))


def _split_body(x_ref, oa_ref, ob_ref):
    oa_ref[...] = x_ref[..., :_W]
    ob_ref[...] = x_ref[..., _W:]


def _slice64(x):
    full = pl.BlockSpec((_BR, _D), lambda i: (i, 0))
    half = pl.BlockSpec((_BR, _W), lambda i: (i, 0))
    return pl.pallas_call(
        _split_body,
        grid=(_NB,),
        in_specs=[full],
        out_specs=[half, half],
        out_shape=[jax.ShapeDtypeStruct((_N, _W), jnp.float32)] * 2,
    )(x)


def _cat_body(a_ref, b_ref, o_ref):
    o_ref[...] = jnp.concatenate([a_ref[...], b_ref[...]], axis=1)


def _unslice(ya, yb):
    full = pl.BlockSpec((_BR, _D), lambda i: (i, 0))
    half = pl.BlockSpec((_BR, _W), lambda i: (i, 0))
    return pl.pallas_call(
        _cat_body,
        grid=(_NB,),
        in_specs=[half, half],
        out_specs=full,
        out_shape=jax.ShapeDtypeStruct((_N, _D), jnp.float32),
    )(ya, yb)


def _softmax_sl_body(a_ref, b_ref, oa_ref, ob_ref):
    x = jnp.concatenate([a_ref[...], b_ref[...]], axis=1)
    m = jnp.max(x, axis=1, keepdims=True)
    e = jnp.exp(x - m)
    s = e / jnp.sum(e, axis=1, keepdims=True)
    oa_ref[...] = s[:, :_W]
    ob_ref[...] = s[:, _W:]


def _softmax_sl(a, b):
    spec = pl.BlockSpec((_BR, _W), lambda i: (i, 0))
    return pl.pallas_call(
        _softmax_sl_body,
        grid=(_NB,),
        in_specs=[spec, spec],
        out_specs=[spec, spec],
        out_shape=[jax.ShapeDtypeStruct((_N, _W), jnp.float32)] * 2,
    )(a, b)


def _l2n(x):
    n = jnp.sqrt(jnp.sum(x * x, axis=1, keepdims=True))
    return x / jnp.clip(n, 1e-12, None)


def _combine_user_body(e0_ref, e1a_ref, e1b_ref, e2a_ref, e2b_ref,
                       m1_ref, m2_ref, o_ref):
    e1 = jnp.concatenate([e1a_ref[...], e1b_ref[...]], axis=1)
    e2 = jnp.concatenate([e2a_ref[...], e2b_ref[...]], axis=1)
    mean = (e0_ref[...] + e1 + e2) * (1.0 / 3.0)
    o_ref[...] = mean + 0.55 * _l2n(m1_ref[...]) + 0.55 * _l2n(m2_ref[...])


def _combine_item_body(e0_ref, e1a_ref, e1b_ref, p2a_ref, p2b_ref,
                       m1_ref, m2_ref, o_ref):
    e1 = jnp.concatenate([e1a_ref[...], e1b_ref[...]], axis=1)
    x = jnp.concatenate([p2a_ref[...], p2b_ref[...]], axis=1)
    m = jnp.max(x, axis=1, keepdims=True)
    e = jnp.exp(x - m)
    e2 = e / jnp.sum(e, axis=1, keepdims=True)
    mean = (e0_ref[...] + e1 + e2) * (1.0 / 3.0)
    o_ref[...] = mean + 0.55 * _l2n(m1_ref[...]) + 0.55 * _l2n(m2_ref[...])


def _combine(body, e0, e1a, e1b, e2a, e2b, m1, m2):
    half = pl.BlockSpec((_BR, _W), lambda i: (i, 0))
    full = pl.BlockSpec((_BR, _D), lambda i: (i, 0))
    return pl.pallas_call(
        body,
        grid=(_NB,),
        in_specs=[full, half, half, half, half, full, full],
        out_specs=full,
        out_shape=jax.ShapeDtypeStruct((_N, _D), jnp.float32),
    )(e0, e1a, e1b, e2a, e2b, m1, m2)


def kernel(ui_indices, ui_values, user_id_emb, item_id_emb, image_feats,
           text_feats, W_img, b_img, W_txt, b_txt):
    pad = _EP - _E
    rows = jnp.concatenate(
        [ui_indices[0].astype(jnp.int32), jnp.zeros((pad,), jnp.int32)]
    ).reshape(_EP // _CH, _CH)
    cols = jnp.concatenate(
        [ui_indices[1].astype(jnp.int32), jnp.zeros((pad,), jnp.int32)]
    ).reshape(_EP // _CH, _CH)
    vals = jnp.concatenate(
        [ui_values.astype(jnp.float32), jnp.zeros((pad,), jnp.float32)]
    ).reshape(_EP // _CH, _CH)

    img_a, img_b = _slice64(_proj(image_feats, W_img, b_img))
    txt_a, txt_b = _slice64(_proj(text_feats, W_txt, b_txt))
    id_a, id_b = _slice64(item_id_emb)

    item_in = jnp.concatenate([img_a, img_b, txt_a, txt_b, id_a, id_b],
                              axis=0)
    user_out = _spmm6(item_in, cols, rows, vals)
    item_out = _spmm6(user_out, rows, cols, vals)

    image_user_feats = _unslice(user_out[0:_N], user_out[_N:2 * _N])
    text_user_feats = _unslice(user_out[2 * _N:3 * _N], user_out[3 * _N:4 * _N])
    image_item_feats = _unslice(item_out[0:_N], item_out[_N:2 * _N])
    text_item_feats = _unslice(item_out[2 * _N:3 * _N], item_out[3 * _N:4 * _N])

    pre_u2 = _spmm2(item_out[4 * _N:], cols, rows, vals)
    u2a, u2b = _softmax_sl(pre_u2[0:_N], pre_u2[_N:])
    pre_i2 = _spmm2(jnp.concatenate([u2a, u2b], axis=0), rows, cols, vals)

    u_final = _combine(_combine_user_body, user_id_emb,
                       user_out[4 * _N:5 * _N], user_out[5 * _N:], u2a, u2b,
                       image_user_feats, text_user_feats)
    i_final = _combine(_combine_item_body, item_id_emb,
                       item_out[4 * _N:5 * _N], item_out[5 * _N:],
                       pre_i2[0:_N], pre_i2[_N:],
                       image_item_feats, text_item_feats)

    prompt_user = jnp.zeros((_N, _D), jnp.float32)
    prompt_item = jnp.zeros((_N, _D), jnp.float32)
    return (u_final, i_final, image_item_feats, text_item_feats,
            image_user_feats, text_user_feats, u_final, i_final,
            prompt_user, prompt_item, 0.0)


# speedup vs baseline: 3.2844x; 1.0790x over previous
import functools

import jax
import jax.numpy as jnp
from jax import lax
from jax.experimental import pallas as pl
from jax.experimental.pallas import tpu as pltpu
from jax.experimental.pallas import tpu_sc as plsc

_N = 50000
_D = 64
_E = 800000
_W = 32
_AW = 16
_CH = 128
_SUB = 4
_NTILE = 16
_SCE = _SUB * _CH
_NSC = (_E // _NTILE + _SCE - 1) // _SCE
_EPT = _NSC * _SCE
_EP = _EPT * _NTILE
_ROWS_PT = _EPT // _CH
_DR = 400
_NDCH = _N // _DR
_DPT = (_NDCH + _NTILE - 1) // _NTILE


def _make_spmm(S):
    mesh = plsc.VectorSubcoreMesh(core_axis_name="c", subcore_axis_name="s")

    @functools.partial(
        pl.kernel,
        mesh=mesh,
        compiler_params=pltpu.CompilerParams(use_tc_tiling_on_sc=False),
        out_type=jax.ShapeDtypeStruct((S * _N, _W), jnp.float32),
        scratch_types=[
            pltpu.VMEM((2, _SUB, _CH), jnp.int32),
            pltpu.VMEM((2, _SUB, _CH), jnp.int32),
            pltpu.VMEM((2, _SUB, _CH), jnp.int32),
            pltpu.VMEM((2, _SUB, _CH), jnp.float32),
            pltpu.VMEM((2, _CH, _W), jnp.float32),
            pltpu.VMEM((_SUB, _CH, _AW), jnp.float32),
            pltpu.VMEM((_DR, _AW), jnp.float32),
            pltpu.VMEM((_DR, _AW), jnp.float32),
            pltpu.VMEM_SHARED((_N, _AW), jnp.float32),
            pltpu.SemaphoreType.DMA,
            pltpu.SemaphoreType.DMA,
            pltpu.SemaphoreType.DMA,
            pltpu.SemaphoreType.DMA,
            pltpu.SemaphoreType.DMA,
        ],
    )
    def spmm(tab_hbm, gidx_hbm, sidx_hbm, val_hbm, out_hbm,
             gi_v, ga_v, si_v, va_v, rows_v, sbuf_v, zero_v, ostg_v, acc_sh,
             sem_i0, sem_i1, sem_g0, sem_g1, sem_s):
        c = lax.axis_index("c")
        t = lax.axis_index("s")
        hoff = c * _AW
        sem_i = (sem_i0, sem_i1)
        sem_g = (sem_g0, sem_g1)

        def zbody(i, carry):
            zero_v[i, pl.ds(0, 16)] = jnp.zeros((16,), jnp.float32)
            return carry

        lax.fori_loop(0, _DR, zbody, 0)

        rbase = t * _ROWS_PT

        def fire_idx(s, p):
            r = rbase + s * _SUB
            pltpu.async_copy(gidx_hbm.at[pl.ds(r, _SUB)], gi_v.at[p], sem_i[p])
            pltpu.async_copy(sidx_hbm.at[pl.ds(r, _SUB)], si_v.at[p], sem_i[p])
            pltpu.async_copy(val_hbm.at[pl.ds(r, _SUB)], va_v.at[p], sem_i[p])

        def wait_idx(s, p):
            r = rbase + s * _SUB
            pltpu.make_async_copy(gidx_hbm.at[pl.ds(r, _SUB)], gi_v.at[p],
                                  sem_i[p]).wait()
            pltpu.make_async_copy(sidx_hbm.at[pl.ds(r, _SUB)], si_v.at[p],
                                  sem_i[p]).wait()
            pltpu.make_async_copy(val_hbm.at[pl.ds(r, _SUB)], va_v.at[p],
                                  sem_i[p]).wait()

        def adjust(p, soff):
            for j in range(_SUB):
                for g in range(_CH // 16):
                    ga_v[p, j, pl.ds(g * 16, 16)] = (
                        gi_v[p, j, pl.ds(g * 16, 16)] + soff)

        def fire_g(p, j, rp):
            pltpu.async_copy(tab_hbm.at[ga_v.at[p, j]], rows_v.at[rp],
                             sem_g[rp])

        def wait_g(p, j, rp):
            pltpu.make_async_copy(tab_hbm.at[ga_v.at[p, j]], rows_v.at[rp],
                                  sem_g[rp]).wait()

        def process(s, p, soff):
            for j in range(_SUB):
                rp = j % 2
                if j + 1 < _SUB:
                    fire_g(p, j + 1, 1 - rp)
                wait_g(p, j, rp)

                def scale_body(g, carry, j=j, rp=rp):
                    b = g * 16
                    vvec = va_v[p, j, pl.ds(b, 16)]
                    for i in range(16):
                        v = vvec[i]
                        sbuf_v[j, b + i, pl.ds(0, 16)] = (
                            rows_v[rp, b + i, pl.ds(hoff, 16)] * v)
                    return carry

                lax.fori_loop(0, _CH // 16, scale_body, 0)
                pltpu.async_copy(sbuf_v.at[j], acc_sh.at[si_v.at[p, j]],
                                 sem_s, add=True)

            @pl.when(s + 1 < _NSC)
            def _():
                wait_idx(s + 1, 1 - p)
                adjust(1 - p, soff)
                fire_g(1 - p, 0, 0)

            for j in range(_SUB):
                pltpu.make_async_copy(sbuf_v.at[j], acc_sh.at[si_v.at[p, j]],
                                      sem_s).wait()

        def slice_body(sl, carry):
            soff = sl * _N
            for j in range(_DPT):
                ci = t + _NTILE * j

                @pl.when(ci < _NDCH)
                def _(ci=ci):
                    pltpu.sync_copy(zero_v, acc_sh.at[pl.ds(ci * _DR, _DR)])

            plsc.subcore_barrier()

            fire_idx(0, 0)
            wait_idx(0, 0)
            adjust(0, soff)
            fire_g(0, 0, 0)

            def sc_pair(k, carry2):
                s0 = 2 * k
                fire_idx(s0 + 1, 1)
                process(s0, 0, soff)

                @pl.when(s0 + 2 < _NSC)
                def _():
                    fire_idx(s0 + 2, 0)

                process(s0 + 1, 1, soff)
                return carry2

            lax.fori_loop(0, _NSC // 2, sc_pair, 0)
            plsc.subcore_barrier()
            for j in range(_DPT):
                ci = t + _NTILE * j

                @pl.when(ci < _NDCH)
                def _(ci=ci):
                    r0 = ci * _DR
                    pltpu.sync_copy(acc_sh.at[pl.ds(r0, _DR)], ostg_v)
                    pltpu.sync_copy(
                        ostg_v,
                        out_hbm.at[pl.ds(soff + r0, _DR), pl.ds(hoff, _AW)])

            return carry

        lax.fori_loop(0, S, slice_body, 0)

    return spmm


_spmm6 = _make_spmm(6)
_spmm2 = _make_spmm(2)


_BR = 2000
_NB = _N // _BR


def _proj_body(x_ref, w_ref, b_ref, o_ref):
    o_ref[...] = (
        jax.lax.dot_general(x_ref[...], w_ref[...],
                            (((1,), (1,)), ((), ())),
                            preferred_element_type=jnp.float32)
        + b_ref[...]
    )


def _proj(x, w, b):
    n, k = x.shape
    return pl.pallas_call(
        _proj_body,
        grid=(_NB,),
        in_specs=[
            pl.BlockSpec((_BR, k), lambda i: (i, 0)),
            pl.BlockSpec((_D, k), lambda i: (0, 0)),
            pl.BlockSpec((1, _D), lambda i: (0, 0)),
        ],
        out_specs=pl.BlockSpec((_BR, _D), lambda i: (i, 0)),
        out_shape=jax.ShapeDtypeStruct((_N, _D), jnp.float32),
    )(x, w, b.reshape(1, _D))


def _split_body(x_ref, oa_ref, ob_ref):
    oa_ref[...] = x_ref[..., :_W]
    ob_ref[...] = x_ref[..., _W:]


def _slice64(x):
    full = pl.BlockSpec((_BR, _D), lambda i: (i, 0))
    half = pl.BlockSpec((_BR, _W), lambda i: (i, 0))
    return pl.pallas_call(
        _split_body,
        grid=(_NB,),
        in_specs=[full],
        out_specs=[half, half],
        out_shape=[jax.ShapeDtypeStruct((_N, _W), jnp.float32)] * 2,
    )(x)


def _cat_body(a_ref, b_ref, o_ref):
    o_ref[...] = jnp.concatenate([a_ref[...], b_ref[...]], axis=1)


def _unslice(ya, yb):
    full = pl.BlockSpec((_BR, _D), lambda i: (i, 0))
    half = pl.BlockSpec((_BR, _W), lambda i: (i, 0))
    return pl.pallas_call(
        _cat_body,
        grid=(_NB,),
        in_specs=[half, half],
        out_specs=full,
        out_shape=jax.ShapeDtypeStruct((_N, _D), jnp.float32),
    )(ya, yb)


def _softmax_sl_body(a_ref, b_ref, oa_ref, ob_ref):
    x = jnp.concatenate([a_ref[...], b_ref[...]], axis=1)
    m = jnp.max(x, axis=1, keepdims=True)
    e = jnp.exp(x - m)
    s = e / jnp.sum(e, axis=1, keepdims=True)
    oa_ref[...] = s[:, :_W]
    ob_ref[...] = s[:, _W:]


def _softmax_sl(a, b):
    spec = pl.BlockSpec((_BR, _W), lambda i: (i, 0))
    return pl.pallas_call(
        _softmax_sl_body,
        grid=(_NB,),
        in_specs=[spec, spec],
        out_specs=[spec, spec],
        out_shape=[jax.ShapeDtypeStruct((_N, _W), jnp.float32)] * 2,
    )(a, b)


def _l2n(x):
    n = jnp.sqrt(jnp.sum(x * x, axis=1, keepdims=True))
    return x / jnp.clip(n, 1e-12, None)


def _combine_user_body(e0_ref, e1a_ref, e1b_ref, e2a_ref, e2b_ref,
                       m1_ref, m2_ref, o_ref):
    e1 = jnp.concatenate([e1a_ref[...], e1b_ref[...]], axis=1)
    e2 = jnp.concatenate([e2a_ref[...], e2b_ref[...]], axis=1)
    mean = (e0_ref[...] + e1 + e2) * (1.0 / 3.0)
    o_ref[...] = mean + 0.55 * _l2n(m1_ref[...]) + 0.55 * _l2n(m2_ref[...])


def _combine_item_body(e0_ref, e1a_ref, e1b_ref, p2a_ref, p2b_ref,
                       m1_ref, m2_ref, o_ref):
    e1 = jnp.concatenate([e1a_ref[...], e1b_ref[...]], axis=1)
    x = jnp.concatenate([p2a_ref[...], p2b_ref[...]], axis=1)
    m = jnp.max(x, axis=1, keepdims=True)
    e = jnp.exp(x - m)
    e2 = e / jnp.sum(e, axis=1, keepdims=True)
    mean = (e0_ref[...] + e1 + e2) * (1.0 / 3.0)
    o_ref[...] = mean + 0.55 * _l2n(m1_ref[...]) + 0.55 * _l2n(m2_ref[...])


def _combine(body, e0, e1a, e1b, e2a, e2b, m1, m2):
    half = pl.BlockSpec((_BR, _W), lambda i: (i, 0))
    full = pl.BlockSpec((_BR, _D), lambda i: (i, 0))
    return pl.pallas_call(
        body,
        grid=(_NB,),
        in_specs=[full, half, half, half, half, full, full],
        out_specs=full,
        out_shape=jax.ShapeDtypeStruct((_N, _D), jnp.float32),
    )(e0, e1a, e1b, e2a, e2b, m1, m2)


def kernel(ui_indices, ui_values, user_id_emb, item_id_emb, image_feats,
           text_feats, W_img, b_img, W_txt, b_txt):
    pad = _EP - _E
    rows = jnp.concatenate(
        [ui_indices[0].astype(jnp.int32), jnp.zeros((pad,), jnp.int32)]
    ).reshape(_EP // _CH, _CH)
    cols = jnp.concatenate(
        [ui_indices[1].astype(jnp.int32), jnp.zeros((pad,), jnp.int32)]
    ).reshape(_EP // _CH, _CH)
    vals = jnp.concatenate(
        [ui_values.astype(jnp.float32), jnp.zeros((pad,), jnp.float32)]
    ).reshape(_EP // _CH, _CH)

    img_a, img_b = _slice64(_proj(image_feats, W_img, b_img))
    txt_a, txt_b = _slice64(_proj(text_feats, W_txt, b_txt))
    id_a, id_b = _slice64(item_id_emb)

    item_in = jnp.concatenate([img_a, img_b, txt_a, txt_b, id_a, id_b],
                              axis=0)
    user_out = _spmm6(item_in, cols, rows, vals)
    item_out = _spmm6(user_out, rows, cols, vals)

    image_user_feats = _unslice(user_out[0:_N], user_out[_N:2 * _N])
    text_user_feats = _unslice(user_out[2 * _N:3 * _N], user_out[3 * _N:4 * _N])
    image_item_feats = _unslice(item_out[0:_N], item_out[_N:2 * _N])
    text_item_feats = _unslice(item_out[2 * _N:3 * _N], item_out[3 * _N:4 * _N])

    pre_u2 = _spmm2(item_out[4 * _N:], cols, rows, vals)
    u2a, u2b = _softmax_sl(pre_u2[0:_N], pre_u2[_N:])
    pre_i2 = _spmm2(jnp.concatenate([u2a, u2b], axis=0), rows, cols, vals)

    u_final = _combine(_combine_user_body, user_id_emb,
                       user_out[4 * _N:5 * _N], user_out[5 * _N:], u2a, u2b,
                       image_user_feats, text_user_feats)
    i_final = _combine(_combine_item_body, item_id_emb,
                       item_out[4 * _N:5 * _N], item_out[5 * _N:],
                       pre_i2[0:_N], pre_i2[_N:],
                       image_item_feats, text_item_feats)

    prompt_user = jnp.zeros((_N, _D), jnp.float32)
    prompt_item = jnp.zeros((_N, _D), jnp.float32)
    return (u_final, i_final, image_item_feats, text_item_feats,
            image_user_feats, text_user_feats, u_final, i_final,
            prompt_user, prompt_item, 0.0)
